# Initial kernel scaffold; baseline (speedup 1.0000x reference)
import jax
import jax.numpy as jnp
from jax.experimental import pallas as pl


def kernel(x_cat0, x_cat1, x_cat2, x_cat3, x_num0, x_num1, table0, table1, table2, table3, gamma0, beta0, W0, b0, gamma1, beta1, W1, b1):
    raise NotImplementedError("write your pallas kernel here")


# speedup vs baseline: 1.8271x; 1.8271x over previous
import functools

import jax
import jax.numpy as jnp
from jax import lax
from jax.experimental import pallas as pl
from jax.experimental.pallas import tpu as pltpu
from jax.experimental.pallas import tpu_sc as plsc

D_MODEL = 128
BATCH = 4096
NUM_DIM = 512

_NC = 2
_NS = 16
_NW = _NC * _NS
_BPW = BATCH // _NW


def _gather_sc(idx_stack, t0, t1, t2, t3):
    mesh = plsc.VectorSubcoreMesh(core_axis_name="c", subcore_axis_name="s")

    @functools.partial(
        pl.kernel,
        mesh=mesh,
        out_type=jax.ShapeDtypeStruct((4, BATCH, D_MODEL), jnp.float32),
        scratch_types=[
            pltpu.VMEM((4, _BPW), jnp.int32),
            pltpu.VMEM((4, _BPW, D_MODEL), jnp.float32),
            pltpu.SemaphoreType.DMA,
        ],
    )
    def body(idx_hbm, tb0, tb1, tb2, tb3, out_hbm, idx_v, rows_v, sem):
        wid = lax.axis_index("s") * _NC + lax.axis_index("c")
        base = wid * _BPW
        pltpu.sync_copy(idx_hbm.at[:, pl.ds(base, _BPW)], idx_v)
        copies = []
        for i, tbl in enumerate((tb0, tb1, tb2, tb3)):
            copies.append(pltpu.async_copy(tbl.at[idx_v.at[i]], rows_v.at[i], sem))
        for i, c in enumerate(copies):
            c.wait()
            pltpu.sync_copy(rows_v.at[i], out_hbm.at[i, pl.ds(base, _BPW)])

    return body(idx_stack, t0, t1, t2, t3)


def _num_body(x0_ref, x1_ref, g0_ref, be0_ref, w0_ref, b0_ref,
              g1_ref, be1_ref, w1_ref, b1_ref, out_ref):
    for j, (x_ref, g_ref, be_ref, w_ref, b_ref) in enumerate((
            (x0_ref, g0_ref, be0_ref, w0_ref, b0_ref),
            (x1_ref, g1_ref, be1_ref, w1_ref, b1_ref))):
        x = x_ref[...]
        mean = jnp.mean(x, axis=0, keepdims=True)
        xc = x - mean
        var = jnp.mean(xc * xc, axis=0, keepdims=True)
        h = xc * (g_ref[...] * lax.rsqrt(var + 1e-5)) + be_ref[...]
        out_ref[j] = (jnp.dot(h, w_ref[...], preferred_element_type=jnp.float32)
                      + b_ref[...])


def kernel(x_cat0, x_cat1, x_cat2, x_cat3, x_num0, x_num1,
           table0, table1, table2, table3,
           gamma0, beta0, W0, b0, gamma1, beta1, W1, b1):
    idx_stack = jnp.stack(
        [x_cat0, x_cat1, x_cat2, x_cat3]).astype(jnp.int32)

    cat_out = _gather_sc(idx_stack, table0, table1, table2, table3)

    num_out = pl.pallas_call(
        _num_body,
        out_shape=jax.ShapeDtypeStruct((2, BATCH, D_MODEL), jnp.float32),
    )(x_num0, x_num1,
      gamma0.reshape(1, NUM_DIM), beta0.reshape(1, NUM_DIM), W0,
      b0.reshape(1, D_MODEL),
      gamma1.reshape(1, NUM_DIM), beta1.reshape(1, NUM_DIM), W1,
      b1.reshape(1, D_MODEL))

    return jnp.concatenate([cat_out, num_out], axis=0)


# speedup vs baseline: 1.9154x; 1.0483x over previous
import functools

import jax
import jax.numpy as jnp
from jax import lax
from jax.experimental import pallas as pl
from jax.experimental.pallas import tpu as pltpu
from jax.experimental.pallas import tpu_sc as plsc

D_MODEL = 128
BATCH = 4096
NUM_DIM = 512

_NC = 2
_NS = 16
_NW = _NC * _NS
_BPW = BATCH // _NW


def _gather_sc(idx_stack, t0, t1, t2, t3):
    mesh = plsc.VectorSubcoreMesh(core_axis_name="c", subcore_axis_name="s")

    @functools.partial(
        pl.kernel,
        mesh=mesh,
        out_type=jax.ShapeDtypeStruct((6, BATCH, D_MODEL), jnp.float32),
        scratch_types=[
            pltpu.VMEM((4, _BPW), jnp.int32),
            pltpu.VMEM((4, _BPW, D_MODEL), jnp.float32),
            pltpu.SemaphoreType.DMA,
        ],
    )
    def body(idx_hbm, tb0, tb1, tb2, tb3, out_hbm, idx_v, rows_v, sem):
        wid = lax.axis_index("s") * _NC + lax.axis_index("c")
        base = wid * _BPW
        pltpu.sync_copy(idx_hbm.at[:, pl.ds(base, _BPW)], idx_v)
        copies = []
        for i, tbl in enumerate((tb0, tb1, tb2, tb3)):
            copies.append(pltpu.async_copy(tbl.at[idx_v.at[i]], rows_v.at[i], sem))
        for i, c in enumerate(copies):
            c.wait()
            pltpu.sync_copy(rows_v.at[i], out_hbm.at[i, pl.ds(base, _BPW)])

    return body(idx_stack, t0, t1, t2, t3)


def _num_body(buf_ref, x0_ref, x1_ref, g_ref, be_ref, w_ref, b_ref, out_ref):
    j = pl.program_id(0)

    def compute(x):
        mean = jnp.mean(x, axis=0, keepdims=True)
        xc = x - mean
        var = jnp.mean(xc * xc, axis=0, keepdims=True)
        h = xc * (g_ref[0] * lax.rsqrt(var + 1e-5)) + be_ref[0]
        out_ref[0] = (jnp.dot(h, w_ref[0], preferred_element_type=jnp.float32)
                      + b_ref[0])

    @pl.when(j == 0)
    def _():
        compute(x0_ref[...])

    @pl.when(j == 1)
    def _():
        compute(x1_ref[...])


def kernel(x_cat0, x_cat1, x_cat2, x_cat3, x_num0, x_num1,
           table0, table1, table2, table3,
           gamma0, beta0, W0, b0, gamma1, beta1, W1, b1):
    idx_stack = jnp.stack(
        [x_cat0, x_cat1, x_cat2, x_cat3]).astype(jnp.int32)

    buf = _gather_sc(idx_stack, table0, table1, table2, table3)

    gs = jnp.stack([gamma0, gamma1]).reshape(2, 1, NUM_DIM)
    bes = jnp.stack([beta0, beta1]).reshape(2, 1, NUM_DIM)
    ws = jnp.stack([W0, W1])
    bs = jnp.stack([b0, b1]).reshape(2, 1, D_MODEL)

    return pl.pallas_call(
        _num_body,
        grid=(2,),
        in_specs=[
            pl.BlockSpec(memory_space=pltpu.MemorySpace.HBM),
            pl.BlockSpec((BATCH, NUM_DIM), lambda j: (0, 0)),
            pl.BlockSpec((BATCH, NUM_DIM), lambda j: (0, 0)),
            pl.BlockSpec((1, 1, NUM_DIM), lambda j: (j, 0, 0)),
            pl.BlockSpec((1, 1, NUM_DIM), lambda j: (j, 0, 0)),
            pl.BlockSpec((1, NUM_DIM, D_MODEL), lambda j: (j, 0, 0)),
            pl.BlockSpec((1, 1, D_MODEL), lambda j: (j, 0, 0)),
        ],
        out_specs=pl.BlockSpec((1, BATCH, D_MODEL), lambda j: (4 + j, 0, 0)),
        out_shape=jax.ShapeDtypeStruct((6, BATCH, D_MODEL), jnp.float32),
        input_output_aliases={0: 0},
    )(buf, x_num0, x_num1, gs, bes, ws, bs)
